# baseline (speedup 1.0000x reference)
import jax
import jax.numpy as jnp
from jax import lax
from jax.experimental import pallas as pl
from jax.experimental.pallas import tpu as pltpu
from jax.experimental.pallas import tpu_sc as plsc

_W = 512
_H = 512
_NBX = 512
_NBY = 512
_BIN_AREA = ((512.0 - 0.0) / _NBX) * ((512.0 - 0.0) / _NBY)

_NC = 2
_NS = 16
_L = 16
_NW = _NC * _NS
_RPW = _W // _NW
_CCH = _H // _L
_BAND = _RPW * _H


def _wcoef(n):
    w0 = jnp.maximum(jnp.minimum(n, 1.0), 0.0)
    w1 = jnp.maximum(jnp.minimum(n - 1.0, 1.0), 0.0)
    return w0, w1


def _sc_body(ns_hbm, site_hbm, o1_hbm, o2_hbm, o3_hbm,
             site_v, ns_v, o1_v, o2_v, o3_v):
    wid = lax.axis_index("s") * _NC + lax.axis_index("c")
    base = wid * _BAND

    pltpu.sync_copy(ns_hbm, ns_v)
    pltpu.sync_copy(site_hbm.at[pl.ds(base, _BAND)], site_v.at[pl.ds(_H, _BAND)])

    @pl.when(wid > 0)
    def _():
        pltpu.sync_copy(site_hbm.at[pl.ds(base - _H, _H)], site_v.at[pl.ds(0, _H)])

    @pl.when(wid == 0)
    def _():
        z = jnp.zeros((_L,), jnp.int32)

        def zero_chunk(i, c):
            site_v[pl.ds(i * _L, _L)] = z
            return c

        lax.fori_loop(0, _CCH, zero_chunk, 0)

    def vgather(x, idx):
        return lax.gather(
            x, idx[:, None],
            lax.GatherDimensionNumbers(
                offset_dims=(), collapsed_slice_dims=(0,), start_index_map=(0,)),
            (1,), mode=lax.GatherScatterMode.PROMISE_IN_BOUNDS)

    nsx = ns_v[pl.ds(0, _L)]
    nsy = ns_v[pl.ds(_L, _L)]
    coefs = []
    for lane in (0, 2, 3):
        lanes = jnp.full((_L,), lane, jnp.int32)
        w0, w1 = _wcoef(vgather(nsx, lanes))
        h0, h1 = _wcoef(vgather(nsy, lanes))
        coefs.append((w0 * h0, w1 * h0, w0 * h1, w1 * h1))

    iota = lax.broadcasted_iota(jnp.int32, (_L,), 0)
    lane0 = iota == 0
    shift_idx = jnp.maximum(iota - 1, 0)
    last_lane = jnp.full((_L,), _L - 1, jnp.int32)
    outs = (o1_v, o2_v, o3_v)

    def chunk(k, carry):
        s_prev, su_prev = carry
        r = k // _CCH
        v = k % _CCH
        cb = v * _L
        jmf = jnp.where((cb + iota) > 0, 1.0, 0.0)
        off = (r + 1) * _H + cb
        s = site_v[pl.ds(off, _L)]
        s_u = site_v[pl.ds(off - _H, _L)]
        s_l = jnp.where(lane0, vgather(s_prev, last_lane), vgather(s, shift_idx))
        s_ul = jnp.where(lane0, vgather(su_prev, last_lane), vgather(s_u, shift_idx))
        for t in (1, 2, 3):
            c00, c10, c01, c11 = coefs[t - 1]
            m = jnp.where(s == t, 1.0, 0.0)
            m_u = jnp.where(s_u == t, 1.0, 0.0)
            m_l = jnp.where(s_l == t, jmf, 0.0)
            m_ul = jnp.where(s_ul == t, jmf, 0.0)
            outs[t - 1][pl.ds(r * _H + cb, _L)] = _BIN_AREA - (
                c00 * m + c10 * m_u + c01 * m_l + c11 * m_ul)
        return (s, s_u)

    z16 = jnp.zeros((_L,), jnp.int32)
    lax.fori_loop(0, _RPW * _CCH, chunk, (z16, z16))

    pltpu.sync_copy(o1_v, o1_hbm.at[pl.ds(base, _BAND)])
    pltpu.sync_copy(o2_v, o2_hbm.at[pl.ds(base, _BAND)])
    pltpu.sync_copy(o3_v, o3_hbm.at[pl.ds(base, _BAND)])


def _sc_call(site_flat, ns32):
    out = jax.ShapeDtypeStruct((_NBX * _NBY,), jnp.float32)
    f = pl.kernel(
        _sc_body,
        mesh=plsc.VectorSubcoreMesh(core_axis_name="c", subcore_axis_name="s"),
        out_type=(out, out, out),
        scratch_types=[
            pltpu.VMEM(((_RPW + 1) * _H,), jnp.int32),
            pltpu.VMEM((2 * _L,), jnp.float32),
            pltpu.VMEM((_BAND,), jnp.float32),
            pltpu.VMEM((_BAND,), jnp.float32),
            pltpu.VMEM((_BAND,), jnp.float32),
        ],
    )
    return f(ns32, site_flat)


def kernel(site_type_map, node_size_x, node_size_y):
    ns32 = jnp.concatenate([
        jnp.pad(node_size_x.astype(jnp.float32), (0, _L - 4)),
        jnp.pad(node_size_y.astype(jnp.float32), (0, _L - 4)),
    ])
    a, b, c = _sc_call(site_type_map.reshape(-1), ns32)
    a = a.reshape(_NBX, _NBY)
    b = b.reshape(_NBX, _NBY)
    c = c.reshape(_NBX, _NBY)
    return (a, a, a, a, a, b, c)


# speedup vs baseline: 1.0430x; 1.0430x over previous
import jax
import jax.numpy as jnp
from jax import lax
from jax.experimental import pallas as pl
from jax.experimental.pallas import tpu as pltpu
from jax.experimental.pallas import tpu_sc as plsc

_W = 512
_H = 512
_NBX = 512
_NBY = 512
_BIN_AREA = ((512.0 - 0.0) / _NBX) * ((512.0 - 0.0) / _NBY)

_NC = 2
_NS = 16
_L = 16
_NW = _NC * _NS
_RPW = _W // _NW
_CCH = _H // _L
_BAND = _RPW * _H
_PAD = 8


def _wcoef(n):
    w0 = jnp.maximum(jnp.minimum(n, 1.0), 0.0)
    w1 = jnp.maximum(jnp.minimum(n - 1.0, 1.0), 0.0)
    return w0, w1


def _sc_body(ns_hbm, site_hbm, o1_hbm, o2_hbm, o3_hbm,
             site_v, ns_v, o1_v, o2_v, o3_v):
    wid = lax.axis_index("s") * _NC + lax.axis_index("c")
    base = wid * _BAND

    pltpu.sync_copy(ns_hbm, ns_v)
    pltpu.sync_copy(site_hbm.at[pl.ds(base, _BAND)],
                    site_v.at[pl.ds(_PAD + _H, _BAND)])

    @pl.when(wid > 0)
    def _():
        pltpu.sync_copy(site_hbm.at[pl.ds(base - _H, _H)],
                        site_v.at[pl.ds(_PAD, _H)])

    @pl.when(wid == 0)
    def _():
        z = jnp.zeros((_L,), jnp.int32)

        def zero_chunk(i, c):
            site_v[pl.ds(_PAD + i * _L, _L)] = z
            return c

        lax.fori_loop(0, _CCH, zero_chunk, 0)

    def vgather(x, idx):
        return lax.gather(
            x, idx[:, None],
            lax.GatherDimensionNumbers(
                offset_dims=(), collapsed_slice_dims=(0,), start_index_map=(0,)),
            (1,), mode=lax.GatherScatterMode.PROMISE_IN_BOUNDS)

    nsx = ns_v[pl.ds(0, _L)]
    nsy = ns_v[pl.ds(_L, _L)]
    coefs = []
    for lane in (0, 2, 3):
        lanes = jnp.full((_L,), lane, jnp.int32)
        w0, w1 = _wcoef(vgather(nsx, lanes))
        h0, h1 = _wcoef(vgather(nsy, lanes))
        coefs.append((w0 * h0, w1 * h0, w0 * h1, w1 * h1))

    iota = lax.broadcasted_iota(jnp.int32, (_L,), 0)
    outs = (o1_v, o2_v, o3_v)

    def col_chunk(v, cc):
        cb = v * _L
        jmf = jnp.where((cb + iota) > 0, 1.0, 0.0)

        def masks(off):
            s = site_v[pl.ds(_PAD + off, _L)]
            s_l = site_v[pl.ds(_PAD + off - 1, _L)]
            return tuple(jnp.where(s == t, 1.0, 0.0) for t in (1, 2, 3)), \
                   tuple(jnp.where(s_l == t, jmf, 0.0) for t in (1, 2, 3))

        def row(r, carry):
            m_u, m_ul = carry
            m, m_l = masks((r + 1) * _H + cb)
            for t in (1, 2, 3):
                c00, c10, c01, c11 = coefs[t - 1]
                outs[t - 1][pl.ds(r * _H + cb, _L)] = _BIN_AREA - (
                    c00 * m[t - 1] + c10 * m_u[t - 1]
                    + c01 * m_l[t - 1] + c11 * m_ul[t - 1])
            return (m, m_l)

        lax.fori_loop(0, _RPW, row, masks(cb), unroll=2)
        return cc

    lax.fori_loop(0, _CCH, col_chunk, 0)

    pltpu.sync_copy(o1_v, o1_hbm.at[pl.ds(base, _BAND)])
    pltpu.sync_copy(o2_v, o2_hbm.at[pl.ds(base, _BAND)])
    pltpu.sync_copy(o3_v, o3_hbm.at[pl.ds(base, _BAND)])


def _sc_call(site_flat, ns32):
    out = jax.ShapeDtypeStruct((_NBX * _NBY,), jnp.float32)
    f = pl.kernel(
        _sc_body,
        mesh=plsc.VectorSubcoreMesh(core_axis_name="c", subcore_axis_name="s"),
        out_type=(out, out, out),
        scratch_types=[
            pltpu.VMEM((_PAD + (_RPW + 1) * _H,), jnp.int32),
            pltpu.VMEM((2 * _L,), jnp.float32),
            pltpu.VMEM((_BAND,), jnp.float32),
            pltpu.VMEM((_BAND,), jnp.float32),
            pltpu.VMEM((_BAND,), jnp.float32),
        ],
    )
    return f(ns32, site_flat)


def kernel(site_type_map, node_size_x, node_size_y):
    ns32 = jnp.concatenate([
        jnp.pad(node_size_x.astype(jnp.float32), (0, _L - 4)),
        jnp.pad(node_size_y.astype(jnp.float32), (0, _L - 4)),
    ])
    a, b, c = _sc_call(site_type_map.reshape(-1), ns32)
    a = a.reshape(_NBX, _NBY)
    b = b.reshape(_NBX, _NBY)
    c = c.reshape(_NBX, _NBY)
    return (a, a, a, a, a, b, c)
